# baseline (speedup 1.0000x reference)
import jax
import jax.numpy as jnp
from jax import lax
from jax.experimental import pallas as pl
from jax.experimental.pallas import tpu as pltpu
from jax.experimental.pallas import tpu_sc as plsc

N = 20000
N2 = 20096
C = 8
MAXD = 100
PAD = 128
L = 16
PART = 4
SEG = N2 // PART
NSCH = SEG // L
NCH = PAD // L
NEG = float("-inf")
SCORE_THR = 0.01
NMS_THR = 0.5
EPS = 1e-8
BIG = 2**30


def _extract(vec, lane, iota):
    return jnp.max(jnp.where(iota == lane, vec, NEG))


def _nms_body(x1h, y1h, x2h, y2h, clsh, osc, oidx,
              x1_v, y1_v, x2_v, y2_v, s_v, csc_v, cidx_v, cand_v, cbuf_v,
              spm_cand):
    ci = lax.axis_index("c")
    si = lax.axis_index("s")
    iota = lax.iota(jnp.int32, L)
    lane0 = iota == 0
    klass = ci * (16 // PART) + si // PART
    part = si % PART
    group = (si // PART) * PART
    base = part * SEG

    def _store1(ref, pos, val):
        plsc.store_scatter(ref, [jnp.full((L,), pos, jnp.int32)],
                           jnp.full((L,), val), mask=lane0)

    pltpu.sync_copy(x1h.at[pl.ds(base, SEG)], x1_v)
    pltpu.sync_copy(y1h.at[pl.ds(base, SEG)], y1_v)
    pltpu.sync_copy(x2h.at[pl.ds(base, SEG)], x2_v)
    pltpu.sync_copy(y2h.at[pl.ds(base, SEG)], y2_v)
    pltpu.sync_copy(clsh.at[pl.ds(klass * N2 + base, SEG)], s_v)

    @pl.when(part == 0)
    def _init():
        def _i(i, carry):
            csc_v[pl.ds(i * L, L)] = jnp.full((L,), NEG, jnp.float32)
            cidx_v[pl.ds(i * L, L)] = jnp.zeros((L,), jnp.int32)
            return carry
        lax.fori_loop(0, NCH, _i, 0)

    bv0 = jnp.full((L,), NEG, jnp.float32)
    bi0 = base + iota

    def _sig(i, carry):
        bv, bi = carry
        off = i * L
        lg = s_v[pl.ds(off, L)]
        p = 1.0 / (1.0 + jnp.exp(-lg))
        s = jnp.where(p > SCORE_THR, p, NEG)
        s_v[pl.ds(off, L)] = s
        idxv = base + off + iota
        better = (s > bv) | ((s == bv) & (idxv < bi))
        return jnp.where(better, s, bv), jnp.where(better, idxv, bi)

    bv, bi = plsc.parallel_loop(0, NSCH, 1, unroll=4,
                                carry=(bv0, bi0))(_sig)
    lm0 = jnp.max(bv)
    li0 = jnp.min(jnp.where(bv == lm0, bi, BIG))

    def _step(t, carry):
        li, lm = carry
        lp = jnp.full((L,), li - base, jnp.int32)
        cx1 = plsc.load_gather(x1_v, [lp])
        cy1 = plsc.load_gather(y1_v, [lp])
        cx2 = plsc.load_gather(x2_v, [lp])
        cy2 = plsc.load_gather(y2_v, [lp])
        areav = (cx2 - cx1) * (cy2 - cy1)
        cand = jnp.where(iota == 0, lm,
                         jnp.where(iota == 1, li.astype(jnp.float32),
                                   jnp.where(iota == 2, cx1,
                                             jnp.where(iota == 3, cy1,
                                                       jnp.where(iota == 4, cx2,
                                                                 jnp.where(iota == 5, cy2, areav))))))
        cand_v[pl.ds(0, L)] = cand
        slot = t & 1
        pltpu.sync_copy(cand_v, spm_cand.at[pl.ds((slot * 16 + si) * L, L)])
        plsc.subcore_barrier()
        pltpu.sync_copy(spm_cand.at[pl.ds((slot * 16 + group) * L, PART * L)],
                        cbuf_v)
        plsc.subcore_barrier()
        win = cbuf_v[pl.ds(0, L)]
        ws = _extract(win, 0, iota)
        wif = _extract(win, 1, iota)
        for k in range(1, PART):
            rk = cbuf_v[pl.ds(k * L, L)]
            sk = _extract(rk, 0, iota)
            ik = _extract(rk, 1, iota)
            better = (sk > ws) | ((sk == ws) & (ik < wif))
            win = jnp.where(better, rk, win)
            ws = jnp.where(better, sk, ws)
            wif = jnp.where(better, ik, wif)
        wx1 = _extract(win, 2, iota)
        wy1 = _extract(win, 3, iota)
        wx2 = _extract(win, 4, iota)
        wy2 = _extract(win, 5, iota)
        wa = _extract(win, 6, iota)
        wi = wif.astype(jnp.int32)

        @pl.when(part == 0)
        def _rec():
            _store1(csc_v, t, ws)
            _store1(cidx_v, t, wi)

        wiv = jnp.full((L,), wi, jnp.int32)

        def _sweep(i, c2):
            bv, bi = c2
            off = i * L
            s = s_v[pl.ds(off, L)]
            x1c = x1_v[pl.ds(off, L)]
            y1c = y1_v[pl.ds(off, L)]
            x2c = x2_v[pl.ds(off, L)]
            y2c = y2_v[pl.ds(off, L)]
            ix1 = jnp.maximum(wx1, x1c)
            iy1 = jnp.maximum(wy1, y1c)
            ix2 = jnp.minimum(wx2, x2c)
            iy2 = jnp.minimum(wy2, y2c)
            inter = jnp.maximum(ix2 - ix1, 0.0) * jnp.maximum(iy2 - iy1, 0.0)
            a2 = (x2c - x1c) * (y2c - y1c)
            den = (wa + a2) - inter + EPS
            idxv = base + off + iota
            kill = (inter / den > NMS_THR) | (idxv == wiv)
            s2 = jnp.where(kill, NEG, s)
            s_v[pl.ds(off, L)] = s2
            better = (s2 > bv) | ((s2 == bv) & (idxv < bi))
            return jnp.where(better, s2, bv), jnp.where(better, idxv, bi)

        bv, bi = plsc.parallel_loop(0, NSCH, 1, unroll=4,
                                    carry=(bv0, bi0))(_sweep)
        m = jnp.max(bv)
        nli = jnp.min(jnp.where(bv == m, bi, BIG))
        return nli, m

    lax.fori_loop(0, MAXD, _step, (li0, lm0))

    @pl.when(part == 0)
    def _out():
        pltpu.sync_copy(csc_v, osc.at[pl.ds(klass * PAD, PAD)])
        pltpu.sync_copy(cidx_v, oidx.at[pl.ds(klass * PAD, PAD)])


def _merge_body(osch, oidxh, x1h, y1h, x2h, y2h, t0h, t1h, t2h, r0h, r1h, r2h,
                outb, outs, outl, outr, outt,
                c1_v, c2_v, c3_v, c4_v, msc_v, midx_v,
                sel_sc, sel_lab, sel_idx, valf_v, cidx_v, cval_v, gout_v,
                spm_f, spm_i):
    ci = lax.axis_index("c")
    si = lax.axis_index("s")
    iota = lax.iota(jnp.int32, L)
    lane0 = iota == 0

    def _store1(ref, pos, val):
        plsc.store_scatter(ref, [jnp.full((L,), pos, jnp.int32)],
                           jnp.full((L,), val), mask=lane0)

    @pl.when((ci == 0) & (si == 1))
    def _stage_boxes():
        pltpu.sync_copy(x1h, c1_v)
        pltpu.sync_copy(y1h, c2_v)
        pltpu.sync_copy(x2h, c3_v)
        pltpu.sync_copy(y2h, c4_v)

    @pl.when((ci == 0) & (si == 2))
    def _stage_trans():
        pltpu.sync_copy(t0h, c1_v)
        pltpu.sync_copy(t1h, c2_v)
        pltpu.sync_copy(t2h, c3_v)

    @pl.when((ci == 0) & (si == 3))
    def _stage_rot():
        pltpu.sync_copy(r0h, c1_v)
        pltpu.sync_copy(r1h, c2_v)
        pltpu.sync_copy(r2h, c3_v)

    @pl.when((ci == 0) & (si == 0))
    def _merge():
        pltpu.sync_copy(osch, msc_v)
        pltpu.sync_copy(oidxh, midx_v)
        rowv = jnp.minimum(iota, C - 1)

        def _initm(i, carry):
            sel_sc[pl.ds(i * L, L)] = jnp.full((L,), NEG, jnp.float32)
            sel_lab[pl.ds(i * L, L)] = jnp.zeros((L,), jnp.int32)
            sel_idx[pl.ds(i * L, L)] = jnp.zeros((L,), jnp.int32)
            return carry

        lax.fori_loop(0, PAD // L, _initm, 0)

        def _mstep(t, heads):
            colv = jnp.where(iota < C, jnp.minimum(heads, PAD - 1), PAD - 1)
            flat = rowv * PAD + colv
            vals = plsc.load_gather(msc_v, [flat])
            m = jnp.max(vals)
            lane = jnp.min(jnp.where(vals == m, iota, 99))
            bidx = plsc.load_gather(midx_v, [flat])
            bsel = jnp.max(jnp.where(iota == lane, bidx, -2**31))
            _store1(sel_sc, t, m)
            _store1(sel_lab, t, lane)
            _store1(sel_idx, t, bsel)
            return jnp.where(iota == lane, heads + 1, heads)

        lax.fori_loop(0, MAXD, _mstep, jnp.zeros((L,), jnp.int32))

        def _mask(i, carry):
            off = i * L
            sc = sel_sc[pl.ds(off, L)]
            v = sc > NEG
            sel_sc[pl.ds(off, L)] = jnp.where(v, sc, -1.0)
            lab = sel_lab[pl.ds(off, L)]
            sel_lab[pl.ds(off, L)] = jnp.where(v, lab, -1)
            valf_v[pl.ds(off, L)] = jnp.where(v, 1.0, 0.0)
            return carry

        lax.fori_loop(0, PAD // L, _mask, 0)
        pltpu.sync_copy(sel_sc, outs)
        pltpu.sync_copy(sel_lab, outl)
        pltpu.sync_copy(sel_idx, spm_i)
        pltpu.sync_copy(valf_v, spm_f)

    plsc.subcore_barrier()

    def _gather_cols(cols, out_ref):
        pltpu.sync_copy(spm_i, cidx_v)
        pltpu.sync_copy(spm_f, cval_v)
        for k, col in enumerate(cols):
            def _g(i, carry, col=col, k=k):
                off = i * L
                sidx = cidx_v[pl.ds(off, L)]
                vals = plsc.load_gather(col, [sidx])
                v = cval_v[pl.ds(off, L)] > 0.0
                gout_v[pl.ds(k * PAD + off, L)] = jnp.where(v, vals, -1.0)
                return carry

            lax.fori_loop(0, PAD // L, _g, 0)
        pltpu.sync_copy(gout_v.at[pl.ds(0, len(cols) * PAD)], out_ref)

    @pl.when((ci == 0) & (si == 1))
    def _gather_boxes():
        _gather_cols([c1_v, c2_v, c3_v, c4_v], outb)

    @pl.when((ci == 0) & (si == 2))
    def _gather_trans():
        _gather_cols([c1_v, c2_v, c3_v], outt)

    @pl.when((ci == 0) & (si == 3))
    def _gather_rot():
        _gather_cols([c1_v, c2_v, c3_v], outr)


@jax.jit
def _sc_filter(x1p, y1p, x2p, y2p, clsp,
               x1, y1, x2, y2, t0, t1, t2, r0, r1, r2):
    f32 = jnp.float32
    i32 = jnp.int32
    nms = pl.kernel(
        _nms_body,
        out_type=[
            jax.ShapeDtypeStruct((C * PAD,), f32),
            jax.ShapeDtypeStruct((C * PAD,), i32),
        ],
        mesh=plsc.VectorSubcoreMesh(core_axis_name="c", subcore_axis_name="s"),
        compiler_params=pltpu.CompilerParams(needs_layout_passes=False),
        scratch_types=[
            pltpu.VMEM((SEG,), f32),
            pltpu.VMEM((SEG,), f32),
            pltpu.VMEM((SEG,), f32),
            pltpu.VMEM((SEG,), f32),
            pltpu.VMEM((SEG,), f32),
            pltpu.VMEM((PAD,), f32),
            pltpu.VMEM((PAD,), i32),
            pltpu.VMEM((L,), f32),
            pltpu.VMEM((PART * L,), f32),
            pltpu.VMEM_SHARED((2 * 16 * L,), f32),
        ],
    )
    osc, oidx = nms(x1p, y1p, x2p, y2p, clsp)
    merge = pl.kernel(
        _merge_body,
        out_type=[
            jax.ShapeDtypeStruct((4 * PAD,), f32),
            jax.ShapeDtypeStruct((PAD,), f32),
            jax.ShapeDtypeStruct((PAD,), i32),
            jax.ShapeDtypeStruct((3 * PAD,), f32),
            jax.ShapeDtypeStruct((3 * PAD,), f32),
        ],
        mesh=plsc.VectorSubcoreMesh(core_axis_name="c", subcore_axis_name="s"),
        compiler_params=pltpu.CompilerParams(needs_layout_passes=False),
        scratch_types=[
            pltpu.VMEM((N,), f32),
            pltpu.VMEM((N,), f32),
            pltpu.VMEM((N,), f32),
            pltpu.VMEM((N,), f32),
            pltpu.VMEM((C * PAD,), f32),
            pltpu.VMEM((C * PAD,), i32),
            pltpu.VMEM((PAD,), f32),
            pltpu.VMEM((PAD,), i32),
            pltpu.VMEM((PAD,), i32),
            pltpu.VMEM((PAD,), f32),
            pltpu.VMEM((PAD,), i32),
            pltpu.VMEM((PAD,), f32),
            pltpu.VMEM((4 * PAD,), f32),
            pltpu.VMEM_SHARED((PAD,), f32),
            pltpu.VMEM_SHARED((PAD,), i32),
        ],
    )
    return merge(osc, oidx, x1, y1, x2, y2, t0, t1, t2, r0, r1, r2)


def kernel(bboxes, classification, translation, rotation):
    bx = bboxes[0]
    trn = translation[0]
    rot = rotation[0]
    zpad = jnp.zeros((N2 - N,), jnp.float32)
    lpad = jnp.full((C, N2 - N), -100.0, jnp.float32)
    clsp = jnp.concatenate([classification[0].T, lpad], axis=1).reshape(-1)
    outb, outs, outl, outr, outt = _sc_filter(
        jnp.concatenate([bx[:, 0], zpad]),
        jnp.concatenate([bx[:, 1], zpad]),
        jnp.concatenate([bx[:, 2], zpad]),
        jnp.concatenate([bx[:, 3], zpad]),
        clsp,
        bx[:, 0], bx[:, 1], bx[:, 2], bx[:, 3],
        trn[:, 0], trn[:, 1], trn[:, 2],
        rot[:, 0], rot[:, 1], rot[:, 2],
    )
    boxes_o = outb.reshape(4, PAD).T[:MAXD][None]
    scores_o = outs[:MAXD][None]
    labels_o = outl[:MAXD][None]
    rot_o = outr.reshape(3, PAD).T[:MAXD][None]
    trans_o = outt.reshape(3, PAD).T[:MAXD][None]
    return boxes_o, scores_o, labels_o, rot_o, trans_o


# speedup vs baseline: 1.1020x; 1.1020x over previous
import jax
import jax.numpy as jnp
from jax import lax
from jax.experimental import pallas as pl
from jax.experimental.pallas import tpu as pltpu
from jax.experimental.pallas import tpu_sc as plsc

N = 20000
N2 = 20096
C = 8
MAXD = 100
PAD = 128
L = 16
PART = 4
SEG = N2 // PART
NSCH = SEG // L
NCH = PAD // L
NEG = float("-inf")
SCORE_THR = 0.01
NMS_THR = 0.5
EPS = 1e-8
BIG = 2**30


def _extract(vec, lane, iota):
    return jnp.max(jnp.where(iota == lane, vec, NEG))


def _nms_body(x1h, y1h, x2h, y2h, clsh, osc, oidx,
              x1_v, y1_v, x2_v, y2_v, s_v, csc_v, cidx_v, cand_v, cbuf_v,
              spm_cand):
    ci = lax.axis_index("c")
    si = lax.axis_index("s")
    iota = lax.iota(jnp.int32, L)
    lane0 = iota == 0
    klass = ci * (16 // PART) + si // PART
    part = si % PART
    group = (si // PART) * PART
    base = part * SEG

    def _store1(ref, pos, val):
        plsc.store_scatter(ref, [jnp.full((L,), pos, jnp.int32)],
                           jnp.full((L,), val), mask=lane0)

    pltpu.sync_copy(x1h.at[pl.ds(base, SEG)], x1_v)
    pltpu.sync_copy(y1h.at[pl.ds(base, SEG)], y1_v)
    pltpu.sync_copy(x2h.at[pl.ds(base, SEG)], x2_v)
    pltpu.sync_copy(y2h.at[pl.ds(base, SEG)], y2_v)
    pltpu.sync_copy(clsh.at[pl.ds(klass * N2 + base, SEG)], s_v)

    @pl.when(part == 0)
    def _init():
        def _i(i, carry):
            csc_v[pl.ds(i * L, L)] = jnp.full((L,), NEG, jnp.float32)
            cidx_v[pl.ds(i * L, L)] = jnp.zeros((L,), jnp.int32)
            return carry
        lax.fori_loop(0, NCH, _i, 0)

    bv0 = jnp.full((L,), NEG, jnp.float32)
    bi0 = base + iota

    def _sig(i, carry):
        bv, bi = carry
        off = i * L
        lg = s_v[pl.ds(off, L)]
        p = 1.0 / (1.0 + jnp.exp(-lg))
        s = jnp.where(p > SCORE_THR, p, NEG)
        s_v[pl.ds(off, L)] = s
        idxv = base + off + iota
        better = s > bv
        return jnp.where(better, s, bv), jnp.where(better, idxv, bi)

    bv, bi = lax.fori_loop(0, NSCH, _sig, (bv0, bi0), unroll=4)
    lm0 = jnp.max(bv)
    li0 = jnp.min(jnp.where(bv == lm0, bi, BIG))

    def _step(t, carry):
        li, lm = carry
        lp = jnp.full((L,), li - base, jnp.int32)
        cx1 = plsc.load_gather(x1_v, [lp])
        cy1 = plsc.load_gather(y1_v, [lp])
        cx2 = plsc.load_gather(x2_v, [lp])
        cy2 = plsc.load_gather(y2_v, [lp])
        areav = (cx2 - cx1) * (cy2 - cy1)
        cand = jnp.where(iota == 0, lm,
                         jnp.where(iota == 1, li.astype(jnp.float32),
                                   jnp.where(iota == 2, cx1,
                                             jnp.where(iota == 3, cy1,
                                                       jnp.where(iota == 4, cx2,
                                                                 jnp.where(iota == 5, cy2, areav))))))
        cand_v[pl.ds(0, L)] = cand
        slot = t & 1
        pltpu.sync_copy(cand_v, spm_cand.at[pl.ds((slot * 16 + si) * L, L)])
        plsc.subcore_barrier()
        pltpu.sync_copy(spm_cand.at[pl.ds((slot * 16 + group) * L, PART * L)],
                        cbuf_v)
        plsc.subcore_barrier()
        win = cbuf_v[pl.ds(0, L)]
        ws = _extract(win, 0, iota)
        wif = _extract(win, 1, iota)
        for k in range(1, PART):
            rk = cbuf_v[pl.ds(k * L, L)]
            sk = _extract(rk, 0, iota)
            ik = _extract(rk, 1, iota)
            better = (sk > ws) | ((sk == ws) & (ik < wif))
            win = jnp.where(better, rk, win)
            ws = jnp.where(better, sk, ws)
            wif = jnp.where(better, ik, wif)
        wx1 = _extract(win, 2, iota)
        wy1 = _extract(win, 3, iota)
        wx2 = _extract(win, 4, iota)
        wy2 = _extract(win, 5, iota)
        wa = _extract(win, 6, iota)
        wi = wif.astype(jnp.int32)

        @pl.when(part == 0)
        def _rec():
            _store1(csc_v, t, ws)
            _store1(cidx_v, t, wi)

        wiv = jnp.full((L,), wi, jnp.int32)

        def _sweep(i, c2):
            bv, bi = c2
            off = i * L
            s = s_v[pl.ds(off, L)]
            x1c = x1_v[pl.ds(off, L)]
            y1c = y1_v[pl.ds(off, L)]
            x2c = x2_v[pl.ds(off, L)]
            y2c = y2_v[pl.ds(off, L)]
            ix1 = jnp.maximum(wx1, x1c)
            iy1 = jnp.maximum(wy1, y1c)
            ix2 = jnp.minimum(wx2, x2c)
            iy2 = jnp.minimum(wy2, y2c)
            inter = jnp.maximum(ix2 - ix1, 0.0) * jnp.maximum(iy2 - iy1, 0.0)
            a2 = (x2c - x1c) * (y2c - y1c)
            den = (wa + a2) - inter + EPS
            idxv = base + off + iota
            kill = (inter / den > NMS_THR) | (idxv == wiv)
            s2 = jnp.where(kill, NEG, s)
            s_v[pl.ds(off, L)] = s2
            better = s2 > bv
            return jnp.where(better, s2, bv), jnp.where(better, idxv, bi)

        bv, bi = lax.fori_loop(0, NSCH, _sweep, (bv0, bi0), unroll=4)
        m = jnp.max(bv)
        nli = jnp.min(jnp.where(bv == m, bi, BIG))
        return nli, m

    lax.fori_loop(0, MAXD, _step, (li0, lm0))

    @pl.when(part == 0)
    def _out():
        pltpu.sync_copy(csc_v, osc.at[pl.ds(klass * PAD, PAD)])
        pltpu.sync_copy(cidx_v, oidx.at[pl.ds(klass * PAD, PAD)])


def _merge_body(osch, oidxh, x1h, y1h, x2h, y2h, t0h, t1h, t2h, r0h, r1h, r2h,
                outb, outs, outl, outr, outt,
                c1_v, c2_v, c3_v, c4_v, msc_v, midx_v,
                sel_sc, sel_lab, sel_idx, valf_v, cidx_v, cval_v, gout_v,
                spm_f, spm_i):
    ci = lax.axis_index("c")
    si = lax.axis_index("s")
    iota = lax.iota(jnp.int32, L)
    lane0 = iota == 0

    def _store1(ref, pos, val):
        plsc.store_scatter(ref, [jnp.full((L,), pos, jnp.int32)],
                           jnp.full((L,), val), mask=lane0)

    @pl.when((ci == 0) & (si == 1))
    def _stage_boxes():
        pltpu.sync_copy(x1h, c1_v)
        pltpu.sync_copy(y1h, c2_v)
        pltpu.sync_copy(x2h, c3_v)
        pltpu.sync_copy(y2h, c4_v)

    @pl.when((ci == 0) & (si == 2))
    def _stage_trans():
        pltpu.sync_copy(t0h, c1_v)
        pltpu.sync_copy(t1h, c2_v)
        pltpu.sync_copy(t2h, c3_v)

    @pl.when((ci == 0) & (si == 3))
    def _stage_rot():
        pltpu.sync_copy(r0h, c1_v)
        pltpu.sync_copy(r1h, c2_v)
        pltpu.sync_copy(r2h, c3_v)

    @pl.when((ci == 0) & (si == 0))
    def _merge():
        pltpu.sync_copy(osch, msc_v)
        pltpu.sync_copy(oidxh, midx_v)
        rowv = jnp.minimum(iota, C - 1)

        def _initm(i, carry):
            sel_sc[pl.ds(i * L, L)] = jnp.full((L,), NEG, jnp.float32)
            sel_lab[pl.ds(i * L, L)] = jnp.zeros((L,), jnp.int32)
            sel_idx[pl.ds(i * L, L)] = jnp.zeros((L,), jnp.int32)
            return carry

        lax.fori_loop(0, PAD // L, _initm, 0)

        def _mstep(t, heads):
            colv = jnp.where(iota < C, jnp.minimum(heads, PAD - 1), PAD - 1)
            flat = rowv * PAD + colv
            vals = plsc.load_gather(msc_v, [flat])
            m = jnp.max(vals)
            lane = jnp.min(jnp.where(vals == m, iota, 99))
            bidx = plsc.load_gather(midx_v, [flat])
            bsel = jnp.max(jnp.where(iota == lane, bidx, -2**31))
            _store1(sel_sc, t, m)
            _store1(sel_lab, t, lane)
            _store1(sel_idx, t, bsel)
            return jnp.where(iota == lane, heads + 1, heads)

        lax.fori_loop(0, MAXD, _mstep, jnp.zeros((L,), jnp.int32))

        def _mask(i, carry):
            off = i * L
            sc = sel_sc[pl.ds(off, L)]
            v = sc > NEG
            sel_sc[pl.ds(off, L)] = jnp.where(v, sc, -1.0)
            lab = sel_lab[pl.ds(off, L)]
            sel_lab[pl.ds(off, L)] = jnp.where(v, lab, -1)
            valf_v[pl.ds(off, L)] = jnp.where(v, 1.0, 0.0)
            return carry

        lax.fori_loop(0, PAD // L, _mask, 0)
        pltpu.sync_copy(sel_sc, outs)
        pltpu.sync_copy(sel_lab, outl)
        pltpu.sync_copy(sel_idx, spm_i)
        pltpu.sync_copy(valf_v, spm_f)

    plsc.subcore_barrier()

    def _gather_cols(cols, out_ref):
        pltpu.sync_copy(spm_i, cidx_v)
        pltpu.sync_copy(spm_f, cval_v)
        for k, col in enumerate(cols):
            def _g(i, carry, col=col, k=k):
                off = i * L
                sidx = cidx_v[pl.ds(off, L)]
                vals = plsc.load_gather(col, [sidx])
                v = cval_v[pl.ds(off, L)] > 0.0
                gout_v[pl.ds(k * PAD + off, L)] = jnp.where(v, vals, -1.0)
                return carry

            lax.fori_loop(0, PAD // L, _g, 0)
        pltpu.sync_copy(gout_v.at[pl.ds(0, len(cols) * PAD)], out_ref)

    @pl.when((ci == 0) & (si == 1))
    def _gather_boxes():
        _gather_cols([c1_v, c2_v, c3_v, c4_v], outb)

    @pl.when((ci == 0) & (si == 2))
    def _gather_trans():
        _gather_cols([c1_v, c2_v, c3_v], outt)

    @pl.when((ci == 0) & (si == 3))
    def _gather_rot():
        _gather_cols([c1_v, c2_v, c3_v], outr)


@jax.jit
def _sc_filter(x1p, y1p, x2p, y2p, clsp,
               x1, y1, x2, y2, t0, t1, t2, r0, r1, r2):
    f32 = jnp.float32
    i32 = jnp.int32
    nms = pl.kernel(
        _nms_body,
        out_type=[
            jax.ShapeDtypeStruct((C * PAD,), f32),
            jax.ShapeDtypeStruct((C * PAD,), i32),
        ],
        mesh=plsc.VectorSubcoreMesh(core_axis_name="c", subcore_axis_name="s"),
        compiler_params=pltpu.CompilerParams(needs_layout_passes=False),
        scratch_types=[
            pltpu.VMEM((SEG,), f32),
            pltpu.VMEM((SEG,), f32),
            pltpu.VMEM((SEG,), f32),
            pltpu.VMEM((SEG,), f32),
            pltpu.VMEM((SEG,), f32),
            pltpu.VMEM((PAD,), f32),
            pltpu.VMEM((PAD,), i32),
            pltpu.VMEM((L,), f32),
            pltpu.VMEM((PART * L,), f32),
            pltpu.VMEM_SHARED((2 * 16 * L,), f32),
        ],
    )
    osc, oidx = nms(x1p, y1p, x2p, y2p, clsp)
    merge = pl.kernel(
        _merge_body,
        out_type=[
            jax.ShapeDtypeStruct((4 * PAD,), f32),
            jax.ShapeDtypeStruct((PAD,), f32),
            jax.ShapeDtypeStruct((PAD,), i32),
            jax.ShapeDtypeStruct((3 * PAD,), f32),
            jax.ShapeDtypeStruct((3 * PAD,), f32),
        ],
        mesh=plsc.VectorSubcoreMesh(core_axis_name="c", subcore_axis_name="s"),
        compiler_params=pltpu.CompilerParams(needs_layout_passes=False),
        scratch_types=[
            pltpu.VMEM((N,), f32),
            pltpu.VMEM((N,), f32),
            pltpu.VMEM((N,), f32),
            pltpu.VMEM((N,), f32),
            pltpu.VMEM((C * PAD,), f32),
            pltpu.VMEM((C * PAD,), i32),
            pltpu.VMEM((PAD,), f32),
            pltpu.VMEM((PAD,), i32),
            pltpu.VMEM((PAD,), i32),
            pltpu.VMEM((PAD,), f32),
            pltpu.VMEM((PAD,), i32),
            pltpu.VMEM((PAD,), f32),
            pltpu.VMEM((4 * PAD,), f32),
            pltpu.VMEM_SHARED((PAD,), f32),
            pltpu.VMEM_SHARED((PAD,), i32),
        ],
    )
    return merge(osc, oidx, x1, y1, x2, y2, t0, t1, t2, r0, r1, r2)


def kernel(bboxes, classification, translation, rotation):
    bx = bboxes[0]
    trn = translation[0]
    rot = rotation[0]
    zpad = jnp.zeros((N2 - N,), jnp.float32)
    lpad = jnp.full((C, N2 - N), -100.0, jnp.float32)
    clsp = jnp.concatenate([classification[0].T, lpad], axis=1).reshape(-1)
    outb, outs, outl, outr, outt = _sc_filter(
        jnp.concatenate([bx[:, 0], zpad]),
        jnp.concatenate([bx[:, 1], zpad]),
        jnp.concatenate([bx[:, 2], zpad]),
        jnp.concatenate([bx[:, 3], zpad]),
        clsp,
        bx[:, 0], bx[:, 1], bx[:, 2], bx[:, 3],
        trn[:, 0], trn[:, 1], trn[:, 2],
        rot[:, 0], rot[:, 1], rot[:, 2],
    )
    boxes_o = outb.reshape(4, PAD).T[:MAXD][None]
    scores_o = outs[:MAXD][None]
    labels_o = outl[:MAXD][None]
    rot_o = outr.reshape(3, PAD).T[:MAXD][None]
    trans_o = outt.reshape(3, PAD).T[:MAXD][None]
    return boxes_o, scores_o, labels_o, rot_o, trans_o
